# Initial kernel scaffold; baseline (speedup 1.0000x reference)
import jax
import jax.numpy as jnp
from jax.experimental import pallas as pl


def kernel(X, g1, g2, g3, params):
    raise NotImplementedError("write your pallas kernel here")


# speedup vs baseline: 1.0023x; 1.0023x over previous
import functools

import jax
import jax.numpy as jnp
from jax.experimental import pallas as pl
from jax.experimental.pallas import tpu as pltpu

NGENE_IN = 4000
NGENE_OUT = 5000
NTOT = 6000
NHIDATT = 64
NHEADS = 4


def _mm_sig_body(a_ref, b_ref, o_ref):
    o_ref[...] = jax.nn.sigmoid(
        jnp.dot(a_ref[...], b_ref[...], preferred_element_type=jnp.float32))


def _matmul_sigmoid_t(a, b):
    m, k = a.shape
    n = b.shape[0]
    bt = b.T
    bm = min(m, 1024)
    bn = min(n, 1024)
    grid = (pl.cdiv(m, bm), pl.cdiv(n, bn))
    return pl.pallas_call(
        _mm_sig_body,
        grid=grid,
        in_specs=[
            pl.BlockSpec((bm, k), lambda i, j: (i, 0)),
            pl.BlockSpec((k, bn), lambda i, j: (0, j)),
        ],
        out_specs=pl.BlockSpec((bm, bn), lambda i, j: (i, j)),
        out_shape=jax.ShapeDtypeStruct((m, n), jnp.float32),
    )(a, bt)


def _mm_bias_body(a_ref, b_ref, bias_ref, o_ref, *, act):
    y = jnp.dot(a_ref[...], b_ref[...], preferred_element_type=jnp.float32)
    y = y + bias_ref[...]
    if act == "elu":
        y = jnp.where(y > 0, y, jnp.exp(jnp.minimum(y, 0.0)) - 1.0)
    o_ref[...] = y


def _matmul_bias(a, w, bias, act=None):
    m, k = a.shape
    n = w.shape[1]
    bm = min(m, 1024)
    grid = (pl.cdiv(m, bm),)
    return pl.pallas_call(
        functools.partial(_mm_bias_body, act=act),
        grid=grid,
        in_specs=[
            pl.BlockSpec((bm, k), lambda i: (i, 0)),
            pl.BlockSpec((k, n), lambda i: (0, 0)),
            pl.BlockSpec((1, n), lambda i: (0, 0)),
        ],
        out_specs=pl.BlockSpec((bm, n), lambda i: (i, 0)),
        out_shape=jax.ShapeDtypeStruct((m, n), jnp.float32),
    )(a, w, bias.reshape(1, n))


def _mlp(layers, x):
    for i, l in enumerate(layers):
        act = "elu" if i < len(layers) - 1 else None
        x = _matmul_bias(x, l["W"], l["b"], act=act)
    return x


def _gat(edge_index, feat, p, heads, fout, n):
    src = edge_index[0]
    dst = edge_index[1]
    h = (feat @ p["W"]).reshape(-1, heads, fout)
    el = jnp.sum(h * p["al"][None], axis=-1)
    er = jnp.sum(h * p["ar"][None], axis=-1)
    e = jax.nn.leaky_relu(el[src] + er[dst], 0.2)
    emax = jax.ops.segment_max(e, dst, num_segments=n)
    emax = jnp.where(jnp.isfinite(emax), emax, 0.0)
    ex = jnp.exp(e - emax[dst])
    den = jax.ops.segment_sum(ex, dst, num_segments=n)
    alpha = ex / (den[dst] + 1e-9)
    out = jax.ops.segment_sum(h[src] * alpha[:, :, None], dst, num_segments=n)
    return (out + p["b"].reshape(1, heads, fout)).reshape(n, heads * fout)


def kernel(X, g1, g2, g3, params):
    z = X[:NGENE_OUT].reshape(-1, 1)
    zp = _gat(g1, z, params["att1"], NHEADS, NHIDATT, NGENE_OUT)
    zp = jax.nn.elu(_gat(g1, zp, params["out1"], 1, NHIDATT, NGENE_OUT))
    zp = _gat(g1, zp, params["att2"], NHEADS, NHIDATT, NGENE_OUT)
    zp = jax.nn.elu(_gat(g1, zp, params["out2"], 1, NHIDATT, NGENE_OUT))
    zp = _gat(g2, zp, params["att3"], NHEADS, NHIDATT, NGENE_OUT)
    zp = jax.nn.elu(_gat(g2, zp, params["out3"], 1, NHIDATT, NGENE_OUT))
    zp = _gat(g2, zp, params["att4"], NHEADS, NHIDATT, NGENE_OUT)
    zp = jax.nn.elu(_gat(g2, zp, params["out4"], 1, NHIDATT, NGENE_OUT))
    g_in_pred = _mlp(params["pred_in"], zp).reshape(-1, 1)[:NGENE_IN, :]
    g_all = jnp.concatenate([g_in_pred, X[NGENE_IN:].reshape(-1, 1)], axis=0)
    zlp = _gat(g3, g_all, params["lp1"], NHEADS, NHIDATT, NTOT)
    zlp = jax.nn.elu(_gat(g3, zlp, params["olp1"], 1, NHIDATT, NTOT))
    zlp = _gat(g3, zlp, params["lp2"], NHEADS, NHIDATT, NTOT)
    zlp = _gat(g3, zlp, params["olp2"], 1, NHIDATT, NTOT)
    zlp = _mlp(params["pred_link"], zlp)
    z1 = zlp[:NGENE_OUT, :]
    z2 = zlp[NGENE_OUT:, :]
    A_semi_ori = _matmul_sigmoid_t(z1, z1)
    A_semi1 = _matmul_sigmoid_t(z1, z2)
    A_semi2 = _matmul_sigmoid_t(z2, z2)
    zs = _gat(g3, g_all, params["s1"], NHEADS, NHIDATT, NTOT)
    zs = jax.nn.elu(_gat(g3, zs, params["os1"], 1, NHIDATT, NTOT))
    zs = _gat(g3, zs, params["s2"], NHEADS, NHIDATT, NTOT)
    zs = _gat(g3, zs, params["os2"], 1, NHIDATT, NTOT)
    g_pred_all = _mlp(params["pred_out"], zs)
    return (g_in_pred, g_pred_all, A_semi1, A_semi2, A_semi_ori)
